# baseline (speedup 1.0000x reference)
import jax
import jax.numpy as jnp
from jax import lax
from jax.experimental import pallas as pl
from jax.experimental.pallas import tpu as pltpu
from jax.experimental.pallas import tpu_sc as plsc

BATCH = 4
SEQ = 2048
WIDTH = 512
NUM_CORES = 2
NUM_SUBCORES = 16
NUM_WORKERS = NUM_CORES * NUM_SUBCORES
S_PER_W = SEQ // NUM_WORKERS
ROWS = 32
NCHUNK = BATCH * S_PER_W // ROWS
LANES = 16
CHUNKS = WIDTH // LANES


def _sc_kernel(x_hbm, pos_hbm, table_hbm, out_hbm,
               idx_v, pos_v, buf0, buf1, buf2, gsem, ssem):
    wid = lax.axis_index("s") * NUM_CORES + lax.axis_index("c")
    s_base = wid * S_PER_W

    for c in range(NCHUNK):
        b, h = divmod(c, NCHUNK // BATCH)
        pltpu.sync_copy(x_hbm.at[pl.ds(b * SEQ + s_base + h * ROWS, ROWS)],
                        idx_v.at[c])

    bufs = (buf0, buf1, buf2)

    def gather(c):
        return pltpu.async_copy(table_hbm.at[idx_v.at[c]], bufs[c % 3], gsem)

    gathers = [gather(0), gather(1)]
    pltpu.sync_copy(pos_hbm.at[pl.ds(s_base, S_PER_W)], pos_v)

    stores = []
    for c in range(NCHUNK):
        b, h = divmod(c, NCHUNK // BATCH)
        buf = bufs[c % 3]
        gathers[c].wait()

        def add_row(i, _, buf=buf, h=h):
            for j in range(CHUNKS):
                sl = pl.ds(j * LANES, LANES)
                buf[i, sl] = buf[i, sl] + pos_v[h * ROWS + i, sl]
            return _

        lax.fori_loop(0, ROWS, add_row, None)
        stores.append(pltpu.async_copy(
            buf, out_hbm.at[pl.ds(b * SEQ + s_base + h * ROWS, ROWS)], ssem))
        if c + 2 < NCHUNK:
            if c >= 1:
                stores[c - 1].wait()
            gathers.append(gather(c + 2))
    for c in range(NCHUNK - 3, NCHUNK):
        stores[c].wait()


@jax.jit
def _gpt_pre_encode(xf, positional_embedding, token_embedding):
    mesh = plsc.VectorSubcoreMesh(core_axis_name="c", subcore_axis_name="s",
                                  num_cores=NUM_CORES,
                                  num_subcores=NUM_SUBCORES)
    run = pl.kernel(
        _sc_kernel,
        out_type=jax.ShapeDtypeStruct((BATCH * SEQ, WIDTH), jnp.float32),
        mesh=mesh,
        scratch_types=[
            pltpu.VMEM((NCHUNK, ROWS), jnp.int32),
            pltpu.VMEM((S_PER_W, WIDTH), jnp.float32),
            pltpu.VMEM((ROWS, WIDTH), jnp.float32),
            pltpu.VMEM((ROWS, WIDTH), jnp.float32),
            pltpu.VMEM((ROWS, WIDTH), jnp.float32),
            pltpu.SemaphoreType.DMA,
            pltpu.SemaphoreType.DMA,
        ],
    )
    return run(xf, positional_embedding, token_embedding)


def kernel(x, positional_embedding, token_embedding):
    xf = x.reshape(BATCH * SEQ).astype(jnp.int32)
    out = _gpt_pre_encode(xf, positional_embedding, token_embedding)
    return out.reshape(BATCH, SEQ, WIDTH)


# speedup vs baseline: 1.1546x; 1.1546x over previous
import jax
import jax.numpy as jnp
from jax import lax
from jax.experimental import pallas as pl
from jax.experimental.pallas import tpu as pltpu
from jax.experimental.pallas import tpu_sc as plsc

BATCH = 4
SEQ = 2048
WIDTH = 512
NUM_CORES = 2
NUM_SUBCORES = 16
NUM_WORKERS = NUM_CORES * NUM_SUBCORES
S_PER_W = SEQ // NUM_WORKERS
ROWS = 32
NCHUNK = BATCH * S_PER_W // ROWS
HALVES = S_PER_W // ROWS
NBUF = 4
LANES = 16
CHUNKS = WIDTH // LANES


def _sc_kernel(x_hbm, pos_hbm, table_hbm, out_hbm,
               idx_v, pos_v, buf0, buf1, buf2, buf3, gsem, ssem, psem):
    wid = lax.axis_index("s") * NUM_CORES + lax.axis_index("c")
    s_base = wid * S_PER_W

    idx_copies = [
        pltpu.async_copy(x_hbm.at[pl.ds(b * SEQ + s_base, S_PER_W)],
                         idx_v.at[b], psem)
        for b in range(BATCH)
    ]
    pos_copy = pltpu.async_copy(pos_hbm.at[pl.ds(s_base, S_PER_W)],
                                pos_v, psem)
    for c in idx_copies:
        c.wait()

    bufs = (buf0, buf1, buf2, buf3)

    def gather(c):
        b, h = divmod(c, HALVES)
        return pltpu.async_copy(
            table_hbm.at[idx_v.at[b, pl.ds(h * ROWS, ROWS)]],
            bufs[c % NBUF], gsem)

    gathers = [gather(0), gather(1)]
    pos_copy.wait()

    stores = []
    for c in range(NCHUNK):
        b, h = divmod(c, HALVES)
        buf = bufs[c % NBUF]
        gathers[c].wait()
        if c + 2 < NCHUNK:
            if c >= 2:
                stores[c - 2].wait()
            gathers.append(gather(c + 2))

        def add_row(i, _, buf=buf, h=h):
            for j in range(CHUNKS):
                sl = pl.ds(j * LANES, LANES)
                buf[i, sl] = buf[i, sl] + pos_v[h * ROWS + i, sl]
            return _

        lax.fori_loop(0, ROWS, add_row, None)
        stores.append(pltpu.async_copy(
            buf, out_hbm.at[pl.ds(b * SEQ + s_base + h * ROWS, ROWS)], ssem))

    for c in range(NCHUNK - 4, NCHUNK):
        stores[c].wait()


@jax.jit
def _gpt_pre_encode(xf, positional_embedding, token_embedding):
    mesh = plsc.VectorSubcoreMesh(core_axis_name="c", subcore_axis_name="s",
                                  num_cores=NUM_CORES,
                                  num_subcores=NUM_SUBCORES)
    run = pl.kernel(
        _sc_kernel,
        out_type=jax.ShapeDtypeStruct((BATCH * SEQ, WIDTH), jnp.float32),
        mesh=mesh,
        scratch_types=[
            pltpu.VMEM((BATCH, S_PER_W), jnp.int32),
            pltpu.VMEM((S_PER_W, WIDTH), jnp.float32),
            pltpu.VMEM((ROWS, WIDTH), jnp.float32),
            pltpu.VMEM((ROWS, WIDTH), jnp.float32),
            pltpu.VMEM((ROWS, WIDTH), jnp.float32),
            pltpu.VMEM((ROWS, WIDTH), jnp.float32),
            pltpu.SemaphoreType.DMA,
            pltpu.SemaphoreType.DMA,
            pltpu.SemaphoreType.DMA,
        ],
    )
    return run(xf, positional_embedding, token_embedding)


def kernel(x, positional_embedding, token_embedding):
    xf = x.reshape(BATCH * SEQ).astype(jnp.int32)
    out = _gpt_pre_encode(xf, positional_embedding, token_embedding)
    return out.reshape(BATCH, SEQ, WIDTH)
